# baseline (speedup 1.0000x reference)
import functools

import jax
import jax.numpy as jnp
from jax import lax
from jax.experimental import pallas as pl
from jax.experimental.pallas import tpu as pltpu
from jax.experimental.pallas import tpu_sc as plsc

D = 64
DP = 128
LANE = 128
G = 2
NBUF = 5
N_WORKERS = 32


def _gather_kernel(n_rows):
    rows_per_w = n_rows // N_WORKERS
    n_chunks = rows_per_w // G
    mesh = plsc.VectorSubcoreMesh(core_axis_name="c", subcore_axis_name="s")

    @functools.partial(
        pl.kernel,
        mesh=mesh,
        out_type=jax.ShapeDtypeStruct((n_rows, LANE, DP), jnp.float32),
        scratch_types=(
            [pltpu.VMEM((rows_per_w, LANE), jnp.int32),
             pltpu.VMEM((NBUF, G, LANE, D), jnp.float32)]
            + [pltpu.SemaphoreType.DMA] * (2 * NBUF)
        ),
        compiler_params=pltpu.CompilerParams(use_tc_tiling_on_sc=False),
    )
    def body(ids_hbm, table_hbm, out_hbm, idx_v, rows_v, *sems):
        gsems = sems[:NBUF]
        ssems = sems[NBUF:]
        wid = lax.axis_index("s") * 2 + lax.axis_index("c")
        base = wid * rows_per_w

        pltpu.sync_copy(ids_hbm.at[pl.ds(base, rows_per_w)], idx_v)

        def fire(k, slot):
            for j in range(G):
                pltpu.async_copy(
                    table_hbm.at[idx_v.at[k * G + j]],
                    rows_v.at[slot, j],
                    gsems[slot],
                )

        def drain_and_store(k, slot):
            for j in range(G):
                pltpu.make_async_copy(
                    table_hbm.at[idx_v.at[k * G + j]],
                    rows_v.at[slot, j],
                    gsems[slot],
                ).wait()
            r0 = base + k * G
            pltpu.async_copy(rows_v.at[slot],
                             out_hbm.at[pl.ds(r0, G), :, pl.ds(0, D)],
                             ssems[slot])

        def wait_store(k, slot):
            r0 = base + k * G
            pltpu.make_async_copy(rows_v.at[slot],
                                  out_hbm.at[pl.ds(r0, G), :, pl.ds(0, D)],
                                  ssems[slot]).wait()

        for k in range(NBUF):
            fire(k, k)
            if k >= 1:
                drain_and_store(k - 1, k - 1)

        def outer(g, carry):
            k0 = g * NBUF
            for b in range(NBUF):
                k = k0 + b
                @pl.when(k - NBUF < n_chunks)
                def _():
                    wait_store(k - NBUF, b)
                @pl.when(k < n_chunks)
                def _():
                    fire(k, b)
                @pl.when(k - 1 < n_chunks)
                def _():
                    drain_and_store(k - 1, (b - 1) % NBUF)
            return carry

        n_groups = -(-(n_chunks + 1 - NBUF) // NBUF)
        lax.fori_loop(1, 1 + n_groups, outer, 0)

        k_last = (1 + n_groups) * NBUF - 1
        for k in range(max(0, k_last - NBUF + 1), n_chunks):
            wait_store(k, k % NBUF)

    return body


def kernel(ids, masks, lengths, extras, table):
    B, L = ids.shape
    n_idx = B * L
    n_rows = n_idx // LANE
    ids2 = (ids * 2).reshape(n_rows, LANE)
    table_pair = jnp.concatenate([table, table], axis=1).reshape(-1, D)
    out = _gather_kernel(n_rows)(ids2, table_pair)
    emb = out.reshape(n_idx, DP)[:, :D].reshape(B, L, D)
    return (emb, masks, lengths, extras)


# speedup vs baseline: 1.1728x; 1.1728x over previous
import functools

import jax
import jax.numpy as jnp
from jax import lax
from jax.experimental import pallas as pl
from jax.experimental.pallas import tpu as pltpu
from jax.experimental.pallas import tpu_sc as plsc

D = 64
DP = 128
LANE = 128
G = 2
NBUF = 5
N_WORKERS = 32


def _gather_kernel(n_rows):
    rows_per_w = n_rows // N_WORKERS
    n_chunks = rows_per_w // G
    mesh = plsc.VectorSubcoreMesh(core_axis_name="c", subcore_axis_name="s")

    @functools.partial(
        pl.kernel,
        mesh=mesh,
        out_type=jax.ShapeDtypeStruct((n_rows, LANE, DP), jnp.float32),
        scratch_types=(
            [pltpu.VMEM((rows_per_w, LANE), jnp.int32),
             pltpu.VMEM((NBUF, G, LANE, D), jnp.float32)]
            + [pltpu.SemaphoreType.DMA] * (2 * NBUF)
        ),
        compiler_params=pltpu.CompilerParams(use_tc_tiling_on_sc=False),
    )
    def body(ids_hbm, table_hbm, out_hbm, idx_v, rows_v, *sems):
        gsems = sems[:NBUF]
        ssems = sems[NBUF:]
        wid = lax.axis_index("s") * 2 + lax.axis_index("c")
        base = wid * rows_per_w

        pltpu.sync_copy(ids_hbm.at[pl.ds(base, rows_per_w)], idx_v)

        def fire(k, slot):
            for j in range(G):
                pltpu.async_copy(
                    table_hbm.at[idx_v.at[k * G + j]],
                    rows_v.at[slot, j],
                    gsems[slot],
                )

        def drain_and_store(k, slot):
            for j in range(G):
                pltpu.make_async_copy(
                    table_hbm.at[idx_v.at[k * G + j]],
                    rows_v.at[slot, j],
                    gsems[slot],
                ).wait()
            r0 = base + k * G
            pltpu.async_copy(rows_v.at[slot],
                             out_hbm.at[pl.ds(r0, G), :, pl.ds(0, D)],
                             ssems[slot])

        def wait_store(k, slot):
            r0 = base + k * G
            pltpu.make_async_copy(rows_v.at[slot],
                                  out_hbm.at[pl.ds(r0, G), :, pl.ds(0, D)],
                                  ssems[slot]).wait()

        for k in range(NBUF):
            fire(k, k)
            if k >= 1:
                drain_and_store(k - 1, k - 1)

        def outer(g, carry):
            k0 = g * NBUF
            for b in range(NBUF):
                k = k0 + b
                @pl.when(k - NBUF < n_chunks)
                def _():
                    wait_store(k - NBUF, b)
                @pl.when(k < n_chunks)
                def _():
                    fire(k, b)
                @pl.when(k - 1 < n_chunks)
                def _():
                    drain_and_store(k - 1, (b - 1) % NBUF)
            return carry

        n_groups = -(-(n_chunks + 1 - NBUF) // NBUF)
        lax.fori_loop(1, 1 + n_groups, outer, 0)

        k_last = (1 + n_groups) * NBUF - 1
        for k in range(max(0, k_last - NBUF + 1), n_chunks):
            wait_store(k, k % NBUF)

    return body


def kernel(ids, masks, lengths, extras, table):
    B, L = ids.shape
    n_idx = B * L
    n_rows = n_idx // LANE
    ids2 = (ids * 2).reshape(n_rows, LANE)
    table_pair = jnp.pad(table, ((0, 0), (0, DP - D))).reshape(-1, D)
    out = _gather_kernel(n_rows)(ids2, table_pair)
    emb = out.reshape(n_idx, DP)[:, :D].reshape(B, L, D)
    return (emb, masks, lengths, extras)
